# Initial kernel scaffold; baseline (speedup 1.0000x reference)
import jax
import jax.numpy as jnp
from jax.experimental import pallas as pl


def kernel(h, segment_ids, num_segments):
    raise NotImplementedError("write your pallas kernel here")


# speedup vs baseline: 4.7738x; 4.7738x over previous
import functools

import jax
import jax.numpy as jnp
from jax import lax
from jax.experimental import pallas as pl
from jax.experimental.pallas import tpu as pltpu
from jax.experimental.pallas import tpu_sc as plsc

N = 320000
D = 128
S = 256
CHUNK = 128
NCHUNKS = N // CHUNK
NC = 2
NS = 16
NW = NC * NS
MAX_ITERS = -(-NCHUNKS // NW)


def _sc_segment_sum(h, seg2d):
    mesh = plsc.VectorSubcoreMesh(core_axis_name="c", subcore_axis_name="s")

    @functools.partial(
        pl.kernel,
        out_type=jax.ShapeDtypeStruct((NC, S, D), jnp.float32),
        mesh=mesh,
        scratch_types=[
            pltpu.VMEM((CHUNK,), jnp.int32),
            pltpu.VMEM((CHUNK, D), jnp.float32),
            pltpu.VMEM((NS, D), jnp.float32),
            pltpu.VMEM_SHARED((S, D), jnp.float32),
            pltpu.SemaphoreType.DMA,
        ],
    )
    def body(h_hbm, seg_hbm, out_hbm, idx_v, rows_v, zero_v, accum_sh, sem):
        cid = lax.axis_index("c")
        sid = lax.axis_index("s")
        wid = sid * NC + cid

        z = jnp.zeros((16,), jnp.float32)
        for r in range(NS):
            for j in range(D // 16):
                zero_v[r, pl.ds(j * 16, 16)] = z
        pltpu.sync_copy(zero_v, accum_sh.at[pl.ds(sid * NS, NS)])
        plsc.subcore_barrier()

        def step(i, carry):
            c = wid + i * NW

            @pl.when(c < NCHUNKS)
            def _():
                pltpu.sync_copy(seg_hbm.at[c], idx_v)
                pltpu.sync_copy(h_hbm.at[pl.ds(c * CHUNK, CHUNK)], rows_v)
                pltpu.async_copy(rows_v, accum_sh.at[idx_v], sem, add=True).wait()

            return carry

        lax.fori_loop(0, MAX_ITERS, step, 0)
        plsc.subcore_barrier()

        pltpu.sync_copy(
            accum_sh.at[pl.ds(sid * NS, NS)],
            out_hbm.at[cid, pl.ds(sid * NS, NS)],
        )

    return body(h, seg2d)


def _combine_body(p_ref, o_ref):
    o_ref[...] = p_ref[0] + p_ref[1]


def kernel(h, segment_ids, num_segments):
    shift = jnp.asarray(num_segments, jnp.int32) - jnp.int32(S)
    seg2d = (segment_ids.astype(jnp.int32) + shift).reshape(NCHUNKS, CHUNK)
    partials = _sc_segment_sum(h, seg2d)
    return pl.pallas_call(
        _combine_body,
        out_shape=jax.ShapeDtypeStruct((S, D), jnp.float32),
    )(partials)


# speedup vs baseline: 8.7165x; 1.8259x over previous
import functools

import jax
import jax.numpy as jnp
from jax import lax
from jax.experimental import pallas as pl
from jax.experimental.pallas import tpu as pltpu
from jax.experimental.pallas import tpu_sc as plsc

N = 320000
D = 128
S = 256
CHUNK = 128
NCHUNKS = N // CHUNK
NC = 2
NS = 16
NW = NC * NS
BLK = 256
CPB = BLK // CHUNK
NBLK = N // BLK
MAX_ITERS = 2 * (-(-NBLK // NW) // 2 + (-(-NBLK // NW)) % 2)


def _sc_segment_sum(h, seg2d):
    mesh = plsc.VectorSubcoreMesh(core_axis_name="c", subcore_axis_name="s")

    @functools.partial(
        pl.kernel,
        out_type=jax.ShapeDtypeStruct((NC, S, D), jnp.float32),
        mesh=mesh,
        scratch_types=[
            pltpu.VMEM((2, CPB, CHUNK), jnp.int32),
            pltpu.VMEM((2, BLK, D), jnp.float32),
            pltpu.VMEM((NS, D), jnp.float32),
            pltpu.VMEM_SHARED((S, D), jnp.float32),
            pltpu.SemaphoreType.DMA,
            pltpu.SemaphoreType.DMA,
        ],
    )
    def body(h_hbm, seg_hbm, out_hbm, idx_v, rows_v, zero_v, accum_sh,
             sem0, sem1):
        cid = lax.axis_index("c")
        sid = lax.axis_index("s")
        wid = sid * NC + cid
        sems = (sem0, sem1)

        def start_load(blk, b, sem):
            pltpu.async_copy(h_hbm.at[pl.ds(blk * BLK, BLK)], rows_v.at[b], sem)
            pltpu.async_copy(seg_hbm.at[pl.ds(blk * CPB, CPB)], idx_v.at[b], sem)

        def wait_load(blk, b, sem):
            pltpu.make_async_copy(
                h_hbm.at[pl.ds(blk * BLK, BLK)], rows_v.at[b], sem).wait()
            pltpu.make_async_copy(
                seg_hbm.at[pl.ds(blk * CPB, CPB)], idx_v.at[b], sem).wait()

        start_load(wid, 0, sems[0])

        z = jnp.zeros((16,), jnp.float32)
        for r in range(NS):
            for j in range(D // 16):
                zero_v[r, pl.ds(j * 16, 16)] = z
        pltpu.sync_copy(zero_v, accum_sh.at[pl.ds(sid * NS, NS)])
        plsc.subcore_barrier()

        def outer(o, carry):
            for b in range(2):
                i = o * 2 + b
                c = wid + i * NW

                @pl.when(c < NBLK)
                def _():
                    wait_load(c, b, sems[b])
                    cn = c + NW

                    @pl.when(cn < NBLK)
                    def _():
                        start_load(cn, 1 - b, sems[1 - b])

                    for j in range(CPB):
                        pltpu.sync_copy(
                            rows_v.at[b, pl.ds(j * CHUNK, CHUNK)],
                            accum_sh.at[idx_v.at[b, j]],
                            add=True,
                        )

            return carry

        lax.fori_loop(0, MAX_ITERS // 2, outer, 0)
        plsc.subcore_barrier()

        pltpu.sync_copy(
            accum_sh.at[pl.ds(sid * NS, NS)],
            out_hbm.at[cid, pl.ds(sid * NS, NS)],
        )

    return body(h, seg2d)


def _combine_body(p_ref, o_ref):
    o_ref[...] = p_ref[0] + p_ref[1]


def kernel(h, segment_ids, num_segments):
    shift = jnp.asarray(num_segments, jnp.int32) - jnp.int32(S)
    seg2d = (segment_ids.astype(jnp.int32) + shift).reshape(NCHUNKS, CHUNK)
    partials = _sc_segment_sum(h, seg2d)
    return pl.pallas_call(
        _combine_body,
        out_shape=jax.ShapeDtypeStruct((S, D), jnp.float32),
    )(partials)
